# baseline (speedup 1.0000x reference)
import functools

import jax
import jax.numpy as jnp
from jax import lax
from jax.experimental import pallas as pl
from jax.experimental.pallas import tpu as pltpu
from jax.experimental.pallas import tpu_sc as plsc

D = 1024
NC = 2
NS = 16
NW = NC * NS
R = 4
S = 8192
B = R * S
B_PER_W = B // NW
WPR = S // B_PER_W
C = 8
NCHUNK = B_PER_W // C
NBUF = 8
LEAD = NBUF // 2
SL = 4


def _build():
    mesh = plsc.VectorSubcoreMesh(core_axis_name="c", subcore_axis_name="s")

    @functools.partial(
        pl.kernel,
        mesh=mesh,
        out_type=jax.ShapeDtypeStruct((B, D), jnp.float32),
        scratch_types=[
            pltpu.VMEM((B_PER_W,), jnp.int32),
            pltpu.VMEM((NBUF, C, D), jnp.float32),
            pltpu.VMEM_SHARED((NS, SL, C, D), jnp.float32),
            pltpu.SemaphoreType.DMA((NBUF,)),
            pltpu.SemaphoreType.DMA((SL,)),
            pltpu.SemaphoreType.DMA((SL,)),
            pltpu.SemaphoreType.DMA,
        ],
    )
    def emb(table_hbm, idx_hbm, out_hbm, idx_v, rows_v, spm, gsem, ssem,
            hsem, isem):
        cid = lax.axis_index("c")
        sid = lax.axis_index("s")
        wid = sid * NC + cid
        base = wid * B_PER_W

        idx_src = idx_hbm.at[wid // WPR, pl.ds((wid % WPR) * B_PER_W, B_PER_W)]
        pltpu.make_async_copy(idx_src, idx_v, isem).start()
        pltpu.make_async_copy(idx_src, idx_v, isem).wait()

        def gather(c, b):
            return pltpu.make_async_copy(
                table_hbm.at[idx_v.at[pl.ds(c * C, C)]], rows_v.at[b], gsem.at[b]
            )

        def to_spmem(b, s):
            return pltpu.make_async_copy(
                rows_v.at[b], spm.at[sid, s], ssem.at[s]
            )

        def to_hbm(c, s):
            return pltpu.make_async_copy(
                spm.at[sid, s], out_hbm.at[pl.ds(base + c * C, C)], hsem.at[s]
            )

        for b in range(LEAD):
            gather(b, b).start()

        def loop_body(i, carry):
            for b in range(NBUF):
                c = i * NBUF + b
                s = c % SL
                sp = (c - 2) % SL
                bn = (b + LEAD) % NBUF

                gather(c, b).wait()

                @pl.when(c >= SL)
                def _():
                    to_hbm(c - SL, s).wait()

                to_spmem(b, s).start()

                @pl.when(c >= 2)
                def _():
                    to_spmem((b - 2) % NBUF, sp).wait()
                    to_hbm(c - 2, sp).start()

                @pl.when(c + LEAD < NCHUNK)
                def _():
                    gather(c + LEAD, bn).start()

            return carry

        lax.fori_loop(0, NCHUNK // NBUF, loop_body, 0)

        last = NCHUNK - 1
        for cc in (last - 1, last):
            to_spmem(cc % NBUF, cc % SL).wait()
            to_hbm(cc, cc % SL).start()
        for k in range(SL):
            to_hbm(last - k, (last - k) % SL).wait()

    return emb


_emb = _build()


def kernel(input_ids, table):
    out = _emb(table, input_ids.astype(jnp.int32))
    return out.reshape(input_ids.shape + (D,))


# speedup vs baseline: 1.0151x; 1.0151x over previous
import functools

import jax
import jax.numpy as jnp
from jax import lax
from jax.experimental import pallas as pl
from jax.experimental.pallas import tpu as pltpu
from jax.experimental.pallas import tpu_sc as plsc

D = 1024
NC = 2
NS = 16
NW = NC * NS
R = 4
S = 8192
B = R * S
B_PER_W = B // NW
WPR = S // B_PER_W
C = 8
NCHUNK = B_PER_W // C
NBUF = 8
LEAD = 6
SL = 4


def _build():
    mesh = plsc.VectorSubcoreMesh(core_axis_name="c", subcore_axis_name="s")

    @functools.partial(
        pl.kernel,
        mesh=mesh,
        out_type=jax.ShapeDtypeStruct((B, D), jnp.float32),
        scratch_types=[
            pltpu.VMEM((B_PER_W,), jnp.int32),
            pltpu.VMEM((NBUF, C, D), jnp.float32),
            pltpu.VMEM_SHARED((NS, SL, C, D), jnp.float32),
            pltpu.SemaphoreType.DMA((NBUF,)),
            pltpu.SemaphoreType.DMA((SL,)),
            pltpu.SemaphoreType.DMA((SL,)),
            pltpu.SemaphoreType.DMA,
        ],
    )
    def emb(table_hbm, idx_hbm, out_hbm, idx_v, rows_v, spm, gsem, ssem,
            hsem, isem):
        cid = lax.axis_index("c")
        sid = lax.axis_index("s")
        wid = sid * NC + cid
        base = wid * B_PER_W

        idx_src = idx_hbm.at[wid // WPR, pl.ds((wid % WPR) * B_PER_W, B_PER_W)]
        pltpu.make_async_copy(idx_src, idx_v, isem).start()
        pltpu.make_async_copy(idx_src, idx_v, isem).wait()

        def gather(c, b):
            return pltpu.make_async_copy(
                table_hbm.at[idx_v.at[pl.ds(c * C, C)]], rows_v.at[b], gsem.at[b]
            )

        def to_spmem(b, s):
            return pltpu.make_async_copy(
                rows_v.at[b], spm.at[sid, s], ssem.at[s]
            )

        def to_hbm(c, s):
            return pltpu.make_async_copy(
                spm.at[sid, s], out_hbm.at[pl.ds(base + c * C, C)], hsem.at[s]
            )

        for b in range(LEAD):
            gather(b, b).start()

        def loop_body(i, carry):
            for b in range(NBUF):
                c = i * NBUF + b
                s = c % SL
                sp = (c - 1) % SL
                bn = (b + LEAD) % NBUF

                gather(c, b).wait()

                @pl.when(c >= SL)
                def _():
                    to_hbm(c - SL, s).wait()

                to_spmem(b, s).start()

                @pl.when(c >= 1)
                def _():
                    to_spmem((b - 1) % NBUF, sp).wait()
                    to_hbm(c - 1, sp).start()

                @pl.when(c + LEAD < NCHUNK)
                def _():
                    gather(c + LEAD, bn).start()

            return carry

        lax.fori_loop(0, NCHUNK // NBUF, loop_body, 0)

        last = NCHUNK - 1
        to_spmem(last % NBUF, last % SL).wait()
        to_hbm(last, last % SL).start()
        for k in range(SL):
            to_hbm(last - k, (last - k) % SL).wait()

    return emb


_emb = _build()


def kernel(input_ids, table):
    out = _emb(table, input_ids.astype(jnp.int32))
    return out.reshape(input_ids.shape + (D,))
